# Initial kernel scaffold; baseline (speedup 1.0000x reference)
import jax
import jax.numpy as jnp
from jax.experimental import pallas as pl


def kernel(x, edge_index, W1m, b1m, W1v, b1v, W2m, b2m, W2v, b2v):
    raise NotImplementedError("write your pallas kernel here")


# speedup vs baseline: 11.6012x; 11.6012x over previous
import functools

import jax
import jax.numpy as jnp
from jax import lax
from jax.experimental import pallas as pl
from jax.experimental.pallas import tpu as pltpu
from jax.experimental.pallas import tpu_sc as plsc

N = 10000
D_IN = 128
D_HID = 32
D_OUT = 64
E = 320000

NC = 2
NS = 16
NW = NC * NS
CHUNK = 128
CPT = 80
EPAD = NW * CPT * CHUNK
NROW_PAD = 10112
RPT = NROW_PAD // NS
DEG_W = 16

_mesh = plsc.VectorSubcoreMesh(core_axis_name="c", subcore_axis_name="s")


def _deg_body(col_hbm, ones_hbm, zeros_hbm, out_hbm, idx_v, ones_v, acc):
    c = lax.axis_index("c")
    s = lax.axis_index("s")
    wid = c * NS + s
    pltpu.sync_copy(zeros_hbm, acc.at[pl.ds(s * RPT, RPT)])
    pltpu.sync_copy(ones_hbm, ones_v)
    pltpu.sync_copy(col_hbm.at[pl.ds(wid * CPT, CPT)], idx_v)
    plsc.subcore_barrier()

    def step(j, _):
        pltpu.sync_copy(ones_v, acc.at[idx_v.at[j]], add=True)
        return _

    lax.fori_loop(0, CPT, step, None)
    plsc.subcore_barrier()
    pltpu.sync_copy(acc.at[pl.ds(s * RPT, RPT)],
                    out_hbm.at[c, pl.ds(s * RPT, RPT)])


_deg_kernel = pl.kernel(
    _deg_body,
    out_type=jax.ShapeDtypeStruct((NC, NROW_PAD, DEG_W), jnp.float32),
    mesh=_mesh,
    compiler_params=pltpu.CompilerParams(use_tc_tiling_on_sc=False),
    scratch_types=[
        pltpu.VMEM((CPT, CHUNK), jnp.int32),
        pltpu.VMEM((CHUNK, DEG_W), jnp.float32),
        pltpu.VMEM_SHARED((NROW_PAD, DEG_W), jnp.float32),
    ],
)


def _make_prop(D):
    def body(tbl_hbm, row_hbm, col_hbm, out_hbm, idxr_v, idxc_v, rows_v, acc):
        c = lax.axis_index("c")
        s = lax.axis_index("s")
        wid = c * NS + s
        pltpu.sync_copy(tbl_hbm.at[pl.ds(c * NROW_PAD + s * RPT, RPT)],
                        acc.at[pl.ds(s * RPT, RPT)])
        pltpu.sync_copy(row_hbm.at[pl.ds(wid * CPT, CPT)], idxr_v)
        pltpu.sync_copy(col_hbm.at[pl.ds(wid * CPT, CPT)], idxc_v)
        plsc.subcore_barrier()

        def step(j, _):
            pltpu.sync_copy(tbl_hbm.at[idxr_v.at[j]], rows_v)
            pltpu.sync_copy(rows_v, acc.at[idxc_v.at[j]], add=True)
            return _

        lax.fori_loop(0, CPT, step, None)
        plsc.subcore_barrier()
        pltpu.sync_copy(acc.at[pl.ds(s * RPT, RPT)],
                        out_hbm.at[c, pl.ds(s * RPT, RPT)])

    return pl.kernel(
        body,
        out_type=jax.ShapeDtypeStruct((NC, NROW_PAD, D), jnp.float32),
        mesh=_mesh,
        compiler_params=pltpu.CompilerParams(use_tc_tiling_on_sc=False),
        scratch_types=[
            pltpu.VMEM((CPT, CHUNK), jnp.int32),
            pltpu.VMEM((CPT, CHUNK), jnp.int32),
            pltpu.VMEM((CHUNK, D), jnp.float32),
            pltpu.VMEM_SHARED((NROW_PAD, D), jnp.float32),
        ],
    )


_prop64 = _make_prop(2 * D_HID)
_prop128 = _make_prop(2 * D_OUT)


_BLK = 1000
_GRID = N // _BLK


def _dis_from_deg(deg_ref):
    d = deg_ref[...]
    deg = d[0, :, 0:1] + d[1, :, 0:1] + 1.0
    dis2 = 1.0 / deg
    return lax.rsqrt(deg), dis2


def _dense1_body(x_ref, w1m_ref, b1m_ref, w1v_ref, b1v_ref, deg_ref, out_ref):
    dis, dis2 = _dis_from_deg(deg_ref)
    xb = x_ref[...]
    m = jax.nn.relu(jnp.dot(xb, w1m_ref[...],
                            preferred_element_type=jnp.float32) + b1m_ref[...])
    v = jax.nn.relu(jnp.dot(xb, w1v_ref[...],
                            preferred_element_type=jnp.float32) + b1v_ref[...])
    att = jnp.exp(-v)
    ma = m * att
    va = v * att * att
    out_ref[...] = jnp.concatenate([ma * dis, va * dis2], axis=1)


def _dense2_body(acc_ref, w2m_ref, b2m_ref, w2v_ref, b2v_ref, deg_ref, out_ref):
    dis, dis2 = _dis_from_deg(deg_ref)
    a = acc_ref[...]
    asum = a[0] + a[1]
    hm = jax.nn.relu(asum[:, :D_HID] * dis)
    hv = jax.nn.relu(asum[:, D_HID:] * dis2)
    m = jax.nn.relu(jnp.dot(hm, w2m_ref[...],
                            preferred_element_type=jnp.float32) + b2m_ref[...])
    v = jax.nn.relu(jnp.dot(hv, w2v_ref[...],
                            preferred_element_type=jnp.float32) + b2v_ref[...])
    att = jnp.exp(-v)
    ma = m * att
    va = v * att * att
    out_ref[...] = jnp.concatenate([ma * dis, va * dis2], axis=1)


def _final_body(acc_ref, deg_ref, eps_ref, out_ref):
    dis, dis2 = _dis_from_deg(deg_ref)
    a = acc_ref[...]
    asum = a[0] + a[1]
    mean = asum[:, :D_OUT] * dis
    var = asum[:, D_OUT:] * dis2
    out_ref[...] = eps_ref[...] * jnp.sqrt(var + 1e-8) + mean


def _row_block(shape2):
    return pl.BlockSpec((_BLK,) + shape2[1:], lambda i: (i,) + (0,) * (len(shape2) - 1))


_full = lambda shape: pl.BlockSpec(shape, lambda i: (0,) * len(shape))
_deg_spec = pl.BlockSpec((NC, _BLK, DEG_W), lambda i: (0, i, 0))


_dense1 = pl.pallas_call(
    _dense1_body,
    grid=(_GRID,),
    in_specs=[
        _row_block((N, D_IN)),
        _full((D_IN, D_HID)), _full((1, D_HID)),
        _full((D_IN, D_HID)), _full((1, D_HID)),
        _deg_spec,
    ],
    out_specs=_row_block((N, 2 * D_HID)),
    out_shape=jax.ShapeDtypeStruct((N, 2 * D_HID), jnp.float32),
)

_dense2 = pl.pallas_call(
    _dense2_body,
    grid=(_GRID,),
    in_specs=[
        pl.BlockSpec((NC, _BLK, 2 * D_HID), lambda i: (0, i, 0)),
        _full((D_HID, D_OUT)), _full((1, D_OUT)),
        _full((D_HID, D_OUT)), _full((1, D_OUT)),
        _deg_spec,
    ],
    out_specs=_row_block((N, 2 * D_OUT)),
    out_shape=jax.ShapeDtypeStruct((N, 2 * D_OUT), jnp.float32),
)

_final = pl.pallas_call(
    _final_body,
    grid=(_GRID,),
    in_specs=[
        pl.BlockSpec((NC, _BLK, 2 * D_OUT), lambda i: (0, i, 0)),
        _deg_spec,
        _row_block((N, D_OUT)),
    ],
    out_specs=_row_block((N, D_OUT)),
    out_shape=jax.ShapeDtypeStruct((N, D_OUT), jnp.float32),
)


@jax.jit
def kernel(x, edge_index, W1m, b1m, W1v, b1v, W2m, b2m, W2v, b2v):
    row = edge_index[0].astype(jnp.int32)
    col = edge_index[1].astype(jnp.int32)
    pad = EPAD - E
    rowp = jnp.concatenate([row, jnp.full((pad,), N, jnp.int32)]).reshape(-1, CHUNK)
    colp = jnp.concatenate([col, jnp.full((pad,), N, jnp.int32)]).reshape(-1, CHUNK)

    ones_rows = jnp.ones((CHUNK, DEG_W), jnp.float32)
    zero_rows = jnp.zeros((RPT, DEG_W), jnp.float32)
    deg_p = _deg_kernel(colp, ones_rows, zero_rows)

    b1m2 = b1m.reshape(1, D_HID)
    b1v2 = b1v.reshape(1, D_HID)
    b2m2 = b2m.reshape(1, D_OUT)
    b2v2 = b2v.reshape(1, D_OUT)

    t1 = _dense1(x, W1m, b1m2, W1v, b1v2, deg_p)
    t1f = jnp.pad(t1, ((0, 2 * NROW_PAD - N), (0, 0)))
    acc1 = _prop64(t1f, rowp, colp)

    t2 = _dense2(acc1, W2m, b2m2, W2v, b2v2, deg_p)
    t2f = jnp.pad(t2, ((0, 2 * NROW_PAD - N), (0, 0)))
    acc2 = _prop128(t2f, rowp, colp)

    eps = jax.random.normal(jax.random.key(42), (N, D_OUT), dtype=jnp.float32)
    return _final(acc2, deg_p, eps)


# speedup vs baseline: 26.8415x; 2.3137x over previous
import functools

import jax
import jax.numpy as jnp
from jax import lax
from jax.experimental import pallas as pl
from jax.experimental.pallas import tpu as pltpu
from jax.experimental.pallas import tpu_sc as plsc

N = 10000
D_IN = 128
D_HID = 32
D_OUT = 64
E = 320000

NC = 2
NS = 16
NW = NC * NS
CHUNK = 128
CPT = 80
EPAD = NW * CPT * CHUNK
NROW_PAD = 10112
RPT = NROW_PAD // NS
DEG_W = 16

_mesh = plsc.VectorSubcoreMesh(core_axis_name="c", subcore_axis_name="s")


def _deg_body(col_hbm, ones_hbm, zeros_hbm, out_hbm, idx_v, ones_v, acc):
    c = lax.axis_index("c")
    s = lax.axis_index("s")
    wid = c * NS + s
    pltpu.sync_copy(zeros_hbm, acc.at[pl.ds(s * RPT, RPT)])
    pltpu.sync_copy(ones_hbm, ones_v)
    pltpu.sync_copy(col_hbm.at[pl.ds(wid * CPT, CPT)], idx_v)
    plsc.subcore_barrier()

    def step(j, _):
        pltpu.sync_copy(ones_v, acc.at[idx_v.at[j]], add=True)
        return _

    lax.fori_loop(0, CPT, step, None)
    plsc.subcore_barrier()
    pltpu.sync_copy(acc.at[pl.ds(s * RPT, RPT)],
                    out_hbm.at[c, pl.ds(s * RPT, RPT)])


_deg_kernel = pl.kernel(
    _deg_body,
    out_type=jax.ShapeDtypeStruct((NC, NROW_PAD, DEG_W), jnp.float32),
    mesh=_mesh,
    compiler_params=pltpu.CompilerParams(use_tc_tiling_on_sc=False),
    scratch_types=[
        pltpu.VMEM((CPT, CHUNK), jnp.int32),
        pltpu.VMEM((CHUNK, DEG_W), jnp.float32),
        pltpu.VMEM_SHARED((NROW_PAD, DEG_W), jnp.float32),
    ],
)


def _make_prop(D):
    def body(tbl_hbm, row_hbm, col_hbm, out_hbm, idxr_v, idxc_v, rows_v, acc):
        c = lax.axis_index("c")
        s = lax.axis_index("s")
        wid = c * NS + s
        pltpu.sync_copy(tbl_hbm.at[pl.ds(c * NROW_PAD + s * RPT, RPT)],
                        acc.at[pl.ds(s * RPT, RPT)])
        pltpu.sync_copy(row_hbm.at[pl.ds(wid * CPT, CPT)], idxr_v)
        pltpu.sync_copy(col_hbm.at[pl.ds(wid * CPT, CPT)], idxc_v)
        plsc.subcore_barrier()

        def step(j, _):
            pltpu.sync_copy(tbl_hbm.at[idxr_v.at[j]], rows_v)
            pltpu.sync_copy(rows_v, acc.at[idxc_v.at[j]], add=True)
            return _

        lax.fori_loop(0, CPT, step, None)
        plsc.subcore_barrier()
        pltpu.sync_copy(acc.at[pl.ds(s * RPT, RPT)],
                        out_hbm.at[c, pl.ds(s * RPT, RPT)])

    return pl.kernel(
        body,
        out_type=jax.ShapeDtypeStruct((NC, NROW_PAD, D), jnp.float32),
        mesh=_mesh,
        compiler_params=pltpu.CompilerParams(use_tc_tiling_on_sc=False),
        scratch_types=[
            pltpu.VMEM((CPT, CHUNK), jnp.int32),
            pltpu.VMEM((CPT, CHUNK), jnp.int32),
            pltpu.VMEM((CHUNK, D), jnp.float32),
            pltpu.VMEM_SHARED((NROW_PAD, D), jnp.float32),
        ],
    )


_prop64 = _make_prop(2 * D_HID)
_prop128 = _make_prop(2 * D_OUT)


_BLK = 1000
_GRID = N // _BLK


def _dis_from_deg(deg_ref):
    d = deg_ref[...]
    deg = d[0, :, 0:1] + d[1, :, 0:1] + 1.0
    dis2 = 1.0 / deg
    return lax.rsqrt(deg), dis2


def _dense1_body(x_ref, w1m_ref, b1m_ref, w1v_ref, b1v_ref, deg_ref, out_ref):
    dis, dis2 = _dis_from_deg(deg_ref)
    xb = x_ref[...]
    m = jax.nn.relu(jnp.dot(xb, w1m_ref[...],
                            preferred_element_type=jnp.float32) + b1m_ref[...])
    v = jax.nn.relu(jnp.dot(xb, w1v_ref[...],
                            preferred_element_type=jnp.float32) + b1v_ref[...])
    att = jnp.exp(-v)
    ma = m * att
    va = v * att * att
    out_ref[...] = jnp.concatenate([ma * dis, va * dis2], axis=1)


def _dense2_body(acc_ref, w2m_ref, b2m_ref, w2v_ref, b2v_ref, deg_ref, out_ref):
    dis, dis2 = _dis_from_deg(deg_ref)
    a = acc_ref[...]
    asum = a[0] + a[1]
    hm = jax.nn.relu(asum[:, :D_HID] * dis)
    hv = jax.nn.relu(asum[:, D_HID:] * dis2)
    m = jax.nn.relu(jnp.dot(hm, w2m_ref[...],
                            preferred_element_type=jnp.float32) + b2m_ref[...])
    v = jax.nn.relu(jnp.dot(hv, w2v_ref[...],
                            preferred_element_type=jnp.float32) + b2v_ref[...])
    att = jnp.exp(-v)
    ma = m * att
    va = v * att * att
    out_ref[...] = jnp.concatenate([ma * dis, va * dis2], axis=1)


def _final_body(acc_ref, deg_ref, eps_ref, out_ref):
    dis, dis2 = _dis_from_deg(deg_ref)
    a = acc_ref[...]
    asum = a[0] + a[1]
    mean = asum[:, :D_OUT] * dis
    var = asum[:, D_OUT:] * dis2
    out_ref[...] = eps_ref[...] * jnp.sqrt(var + 1e-8) + mean


def _row_block(shape2):
    return pl.BlockSpec((_BLK,) + shape2[1:], lambda i: (i,) + (0,) * (len(shape2) - 1))


_full = lambda shape: pl.BlockSpec(shape, lambda i: (0,) * len(shape))
_deg_spec = pl.BlockSpec((NC, _BLK, DEG_W), lambda i: (0, i, 0))


_dense1 = pl.pallas_call(
    _dense1_body,
    grid=(_GRID,),
    in_specs=[
        _row_block((N, D_IN)),
        _full((D_IN, D_HID)), _full((1, D_HID)),
        _full((D_IN, D_HID)), _full((1, D_HID)),
        _deg_spec,
    ],
    out_specs=_row_block((N, 2 * D_HID)),
    out_shape=jax.ShapeDtypeStruct((N, 2 * D_HID), jnp.float32),
)

_dense2 = pl.pallas_call(
    _dense2_body,
    grid=(_GRID,),
    in_specs=[
        pl.BlockSpec((NC, _BLK, 2 * D_HID), lambda i: (0, i, 0)),
        _full((D_HID, D_OUT)), _full((1, D_OUT)),
        _full((D_HID, D_OUT)), _full((1, D_OUT)),
        _deg_spec,
    ],
    out_specs=_row_block((N, 2 * D_OUT)),
    out_shape=jax.ShapeDtypeStruct((N, 2 * D_OUT), jnp.float32),
)

_final = pl.pallas_call(
    _final_body,
    grid=(_GRID,),
    in_specs=[
        pl.BlockSpec((NC, _BLK, 2 * D_OUT), lambda i: (0, i, 0)),
        _deg_spec,
        _row_block((N, D_OUT)),
    ],
    out_specs=_row_block((N, D_OUT)),
    out_shape=jax.ShapeDtypeStruct((N, D_OUT), jnp.float32),
)


@jax.jit
def kernel(x, edge_index, W1m, b1m, W1v, b1v, W2m, b2m, W2v, b2v):
    row = edge_index[0].astype(jnp.int32)
    col = edge_index[1].astype(jnp.int32)
    pad = EPAD - E
    padidx = N + (jnp.arange(pad, dtype=jnp.int32) % (NROW_PAD - N))
    rowp = jnp.concatenate([row, padidx]).reshape(-1, CHUNK)
    colp = jnp.concatenate([col, padidx]).reshape(-1, CHUNK)

    ones_rows = jnp.ones((CHUNK, DEG_W), jnp.float32)
    zero_rows = jnp.zeros((RPT, DEG_W), jnp.float32)
    deg_p = _deg_kernel(colp, ones_rows, zero_rows)

    b1m2 = b1m.reshape(1, D_HID)
    b1v2 = b1v.reshape(1, D_HID)
    b2m2 = b2m.reshape(1, D_OUT)
    b2v2 = b2v.reshape(1, D_OUT)

    t1 = _dense1(x, W1m, b1m2, W1v, b1v2, deg_p)
    t1f = jnp.pad(t1, ((0, 2 * NROW_PAD - N), (0, 0)))
    acc1 = _prop64(t1f, rowp, colp)

    t2 = _dense2(acc1, W2m, b2m2, W2v, b2v2, deg_p)
    t2f = jnp.pad(t2, ((0, 2 * NROW_PAD - N), (0, 0)))
    acc2 = _prop128(t2f, rowp, colp)

    eps = jax.random.normal(jax.random.key(42), (N, D_OUT), dtype=jnp.float32)
    return _final(acc2, deg_p, eps)


# speedup vs baseline: 32.0959x; 1.1958x over previous
import functools

import jax
import jax.numpy as jnp
from jax import lax
from jax.experimental import pallas as pl
from jax.experimental.pallas import tpu as pltpu
from jax.experimental.pallas import tpu_sc as plsc

N = 10000
D_IN = 128
D_HID = 32
D_OUT = 64
E = 320000

NC = 2
NS = 16
NW = NC * NS
CHUNK = 128
CPT = 80
EPAD = NW * CPT * CHUNK
NROW_PAD = 10112
RPT = NROW_PAD // NS
DEG_W = 16

_mesh = plsc.VectorSubcoreMesh(core_axis_name="c", subcore_axis_name="s")


def _deg_body(col_hbm, ones_hbm, zeros_hbm, out_hbm, idx_v, ones_v, acc):
    c = lax.axis_index("c")
    s = lax.axis_index("s")
    wid = c * NS + s
    pltpu.sync_copy(zeros_hbm, acc.at[pl.ds(s * RPT, RPT)])
    pltpu.sync_copy(ones_hbm, ones_v)
    pltpu.sync_copy(col_hbm.at[pl.ds(wid * CPT, CPT)], idx_v)
    plsc.subcore_barrier()

    def step(j, _):
        pltpu.sync_copy(ones_v, acc.at[idx_v.at[j]], add=True)
        return _

    lax.fori_loop(0, CPT, step, None)
    plsc.subcore_barrier()
    pltpu.sync_copy(acc.at[pl.ds(s * RPT, RPT)],
                    out_hbm.at[c, pl.ds(s * RPT, RPT)])


_deg_kernel = pl.kernel(
    _deg_body,
    out_type=jax.ShapeDtypeStruct((NC, NROW_PAD, DEG_W), jnp.float32),
    mesh=_mesh,
    compiler_params=pltpu.CompilerParams(use_tc_tiling_on_sc=False),
    scratch_types=[
        pltpu.VMEM((CPT, CHUNK), jnp.int32),
        pltpu.VMEM((CHUNK, DEG_W), jnp.float32),
        pltpu.VMEM_SHARED((NROW_PAD, DEG_W), jnp.float32),
    ],
)


IB = 16


def _make_prop(D, nbuf):
    def body(tbl_hbm, row_hbm, col_hbm, out_hbm, idxr_v, idxc_v, *rest):
        bufs = rest[:nbuf]
        gsem = rest[nbuf:2 * nbuf]
        ssem = rest[2 * nbuf:3 * nbuf]
        acc = rest[3 * nbuf]
        c = lax.axis_index("c")
        s = lax.axis_index("s")
        wid = c * NS + s
        pltpu.sync_copy(tbl_hbm.at[pl.ds(c * NROW_PAD + s * RPT, RPT)],
                        acc.at[pl.ds(s * RPT, RPT)])
        plsc.subcore_barrier()

        def blk(kb, carry):
            base = wid * CPT + kb * IB
            pltpu.sync_copy(row_hbm.at[pl.ds(base, IB)], idxr_v)
            pltpu.sync_copy(col_hbm.at[pl.ds(base, IB)], idxc_v)
            for b in range(nbuf):
                pltpu.async_copy(tbl_hbm.at[idxr_v.at[b]], bufs[b], gsem[b])

            def step(it, carry2):
                j0 = it * nbuf
                for b in range(nbuf):
                    j = j0 + b
                    pltpu.make_async_copy(tbl_hbm.at[idxr_v.at[j]],
                                          bufs[b], gsem[b]).wait()
                    pltpu.async_copy(bufs[b], acc.at[idxc_v.at[j]], ssem[b],
                                     add=True)
                for b in range(nbuf):
                    j = j0 + b
                    pltpu.make_async_copy(bufs[b], acc.at[idxc_v.at[j]],
                                          ssem[b]).wait()
                    nxt = j + nbuf

                    @pl.when(nxt < IB)
                    def _():
                        pltpu.async_copy(tbl_hbm.at[idxr_v.at[nxt]],
                                         bufs[b], gsem[b])
                return carry2

            lax.fori_loop(0, IB // nbuf, step, None)
            return carry

        lax.fori_loop(0, CPT // IB, blk, None)
        plsc.subcore_barrier()
        pltpu.sync_copy(acc.at[pl.ds(s * RPT, RPT)],
                        out_hbm.at[c, pl.ds(s * RPT, RPT)])

    return pl.kernel(
        body,
        out_type=jax.ShapeDtypeStruct((NC, NROW_PAD, D), jnp.float32),
        mesh=_mesh,
        compiler_params=pltpu.CompilerParams(use_tc_tiling_on_sc=False),
        scratch_types=[
            pltpu.VMEM((IB, CHUNK), jnp.int32),
            pltpu.VMEM((IB, CHUNK), jnp.int32),
        ] + [pltpu.VMEM((CHUNK, D), jnp.float32)] * nbuf
          + [pltpu.SemaphoreType.DMA] * (2 * nbuf)
          + [pltpu.VMEM_SHARED((NROW_PAD, D), jnp.float32)],
    )


_prop64 = _make_prop(2 * D_HID, 4)
_prop128 = _make_prop(2 * D_OUT, 2)


_BLK = 1000
_GRID = N // _BLK


def _dis_from_deg(deg_ref):
    d = deg_ref[...]
    deg = d[0, :, 0:1] + d[1, :, 0:1] + 1.0
    dis2 = 1.0 / deg
    return lax.rsqrt(deg), dis2


def _dense1_body(x_ref, w1m_ref, b1m_ref, w1v_ref, b1v_ref, deg_ref, out_ref):
    dis, dis2 = _dis_from_deg(deg_ref)
    xb = x_ref[...]
    m = jax.nn.relu(jnp.dot(xb, w1m_ref[...],
                            preferred_element_type=jnp.float32) + b1m_ref[...])
    v = jax.nn.relu(jnp.dot(xb, w1v_ref[...],
                            preferred_element_type=jnp.float32) + b1v_ref[...])
    att = jnp.exp(-v)
    ma = m * att
    va = v * att * att
    out_ref[...] = jnp.concatenate([ma * dis, va * dis2], axis=1)


def _dense2_body(acc_ref, w2m_ref, b2m_ref, w2v_ref, b2v_ref, deg_ref, out_ref):
    dis, dis2 = _dis_from_deg(deg_ref)
    a = acc_ref[...]
    asum = a[0] + a[1]
    hm = jax.nn.relu(asum[:, :D_HID] * dis)
    hv = jax.nn.relu(asum[:, D_HID:] * dis2)
    m = jax.nn.relu(jnp.dot(hm, w2m_ref[...],
                            preferred_element_type=jnp.float32) + b2m_ref[...])
    v = jax.nn.relu(jnp.dot(hv, w2v_ref[...],
                            preferred_element_type=jnp.float32) + b2v_ref[...])
    att = jnp.exp(-v)
    ma = m * att
    va = v * att * att
    out_ref[...] = jnp.concatenate([ma * dis, va * dis2], axis=1)


def _final_body(acc_ref, deg_ref, eps_ref, out_ref):
    dis, dis2 = _dis_from_deg(deg_ref)
    a = acc_ref[...]
    asum = a[0] + a[1]
    mean = asum[:, :D_OUT] * dis
    var = asum[:, D_OUT:] * dis2
    out_ref[...] = eps_ref[...] * jnp.sqrt(var + 1e-8) + mean


def _row_block(shape2):
    return pl.BlockSpec((_BLK,) + shape2[1:], lambda i: (i,) + (0,) * (len(shape2) - 1))


_full = lambda shape: pl.BlockSpec(shape, lambda i: (0,) * len(shape))
_deg_spec = pl.BlockSpec((NC, _BLK, DEG_W), lambda i: (0, i, 0))


_dense1 = pl.pallas_call(
    _dense1_body,
    grid=(_GRID,),
    in_specs=[
        _row_block((N, D_IN)),
        _full((D_IN, D_HID)), _full((1, D_HID)),
        _full((D_IN, D_HID)), _full((1, D_HID)),
        _deg_spec,
    ],
    out_specs=_row_block((N, 2 * D_HID)),
    out_shape=jax.ShapeDtypeStruct((N, 2 * D_HID), jnp.float32),
)

_dense2 = pl.pallas_call(
    _dense2_body,
    grid=(_GRID,),
    in_specs=[
        pl.BlockSpec((NC, _BLK, 2 * D_HID), lambda i: (0, i, 0)),
        _full((D_HID, D_OUT)), _full((1, D_OUT)),
        _full((D_HID, D_OUT)), _full((1, D_OUT)),
        _deg_spec,
    ],
    out_specs=_row_block((N, 2 * D_OUT)),
    out_shape=jax.ShapeDtypeStruct((N, 2 * D_OUT), jnp.float32),
)

_final = pl.pallas_call(
    _final_body,
    grid=(_GRID,),
    in_specs=[
        pl.BlockSpec((NC, _BLK, 2 * D_OUT), lambda i: (0, i, 0)),
        _deg_spec,
        _row_block((N, D_OUT)),
    ],
    out_specs=_row_block((N, D_OUT)),
    out_shape=jax.ShapeDtypeStruct((N, D_OUT), jnp.float32),
)


@jax.jit
def kernel(x, edge_index, W1m, b1m, W1v, b1v, W2m, b2m, W2v, b2v):
    row = edge_index[0].astype(jnp.int32)
    col = edge_index[1].astype(jnp.int32)
    pad = EPAD - E
    padidx = N + (jnp.arange(pad, dtype=jnp.int32) % (NROW_PAD - N))
    rowp = jnp.concatenate([row, padidx]).reshape(-1, CHUNK)
    colp = jnp.concatenate([col, padidx]).reshape(-1, CHUNK)

    ones_rows = jnp.ones((CHUNK, DEG_W), jnp.float32)
    zero_rows = jnp.zeros((RPT, DEG_W), jnp.float32)
    deg_p = _deg_kernel(colp, ones_rows, zero_rows)

    b1m2 = b1m.reshape(1, D_HID)
    b1v2 = b1v.reshape(1, D_HID)
    b2m2 = b2m.reshape(1, D_OUT)
    b2v2 = b2v.reshape(1, D_OUT)

    t1 = _dense1(x, W1m, b1m2, W1v, b1v2, deg_p)
    t1f = jnp.pad(t1, ((0, 2 * NROW_PAD - N), (0, 0)))
    acc1 = _prop64(t1f, rowp, colp)

    t2 = _dense2(acc1, W2m, b2m2, W2v, b2v2, deg_p)
    t2f = jnp.pad(t2, ((0, 2 * NROW_PAD - N), (0, 0)))
    acc2 = _prop128(t2f, rowp, colp)

    eps = jax.random.normal(jax.random.key(42), (N, D_OUT), dtype=jnp.float32)
    return _final(acc2, deg_p, eps)
